# Initial kernel scaffold; baseline (speedup 1.0000x reference)
import jax
import jax.numpy as jnp
from jax.experimental import pallas as pl


def kernel(node_feat, coord_feat, edge_index, edge_feat, W_e1, b_e1, W_e2, b_e2, W_q, W_k, W_n1, b_n1, W_n2, b_n2):
    raise NotImplementedError("write your pallas kernel here")


# speedup vs baseline: 3.4434x; 3.4434x over previous
import dataclasses
import functools
import math

import jax
import jax.numpy as jnp
from jax import lax
from jax.experimental import pallas as pl
from jax.experimental.pallas import tpu as pltpu
from jax.experimental.pallas import tpu_sc as plsc

N = 10000
E = 320000
IN = 128
HID = 128
OUT = 128
EF = 4

NC = 2
NS = 16
NW = NC * NS
EPW = E // NW
GC = 80
NCHUNK = EPW // GC
NP = 10240
SC_ROWS_PER_SUB = NP // NS

TW = 384
SCC = 80

RB = 400
BE = 640


def _silu(x):
    return x * jax.nn.sigmoid(x)


def _tables_body(nf, c8, wcat, be1, wt8, stab, dtab):
    m = jnp.dot(nf[...], wcat[...], preferred_element_type=jnp.float32)
    a = m[:, :HID] + be1[...]
    b = m[:, HID:2 * HID]
    k = m[:, 2 * HID:3 * HID]
    q = m[:, 3 * HID:]
    p8 = jnp.dot(q, wt8[...], preferred_element_type=jnp.float32)
    rows = nf.shape[0]
    zs = jnp.zeros((rows, TW - 2 * HID - 3), jnp.float32)
    stab[...] = jnp.concatenate([a, k, c8[:, :3], zs], axis=1)
    zd = jnp.zeros((rows, TW - 2 * HID - 8), jnp.float32)
    dtab[...] = jnp.concatenate([b, q, c8[:, :3], p8[:, :5], zd], axis=1)


def _make_tables(nf, coord8, wcat, be1, wt8):
    grid = (N // RB,)
    return pl.pallas_call(
        _tables_body,
        grid=grid,
        in_specs=[
            pl.BlockSpec((RB, IN), lambda i: (i, 0)),
            pl.BlockSpec((RB, 8), lambda i: (i, 0)),
            pl.BlockSpec((IN, 4 * HID), lambda i: (0, 0)),
            pl.BlockSpec((1, HID), lambda i: (0, 0)),
            pl.BlockSpec((HID, 8), lambda i: (0, 0)),
        ],
        out_specs=[
            pl.BlockSpec((RB, TW), lambda i: (i, 0)),
            pl.BlockSpec((RB, TW), lambda i: (i, 0)),
        ],
        out_shape=[
            jax.ShapeDtypeStruct((N, TW), jnp.float32),
            jax.ShapeDtypeStruct((N, TW), jnp.float32),
        ],
    )(nf, coord8, wcat, be1, wt8)


def _gather_kernel(stab, dtab, src, dst,
                   gs, gd,
                   idxs, idxd, rs, rd, s0, s1):
    wid = lax.axis_index("s") * NC + lax.axis_index("c")

    @pl.loop(0, NCHUNK)
    def _(k):
        base = wid * EPW + k * GC
        pltpu.sync_copy(src.at[pl.ds(base, GC)], idxs)
        pltpu.sync_copy(dst.at[pl.ds(base, GC)], idxd)
        c0 = pltpu.async_copy(stab.at[idxs], rs, s0)
        c1 = pltpu.async_copy(dtab.at[idxd], rd, s1)
        c0.wait()
        c1.wait()
        pltpu.sync_copy(rs, gs.at[pl.ds(base, GC)])
        pltpu.sync_copy(rd, gd.at[pl.ds(base, GC)])


def _gather(stab, dtab, src, dst):
    mesh = plsc.VectorSubcoreMesh(core_axis_name="c", subcore_axis_name="s")
    kern = pl.kernel(
        _gather_kernel,
        mesh=mesh,
        out_type=[
            jax.ShapeDtypeStruct((E, TW), jnp.float32),
            jax.ShapeDtypeStruct((E, TW), jnp.float32),
        ],
        scratch_types=[
            pltpu.VMEM((GC,), jnp.int32),
            pltpu.VMEM((GC,), jnp.int32),
            pltpu.VMEM((GC, TW), jnp.float32),
            pltpu.VMEM((GC, TW), jnp.float32),
            pltpu.SemaphoreType.DMA,
            pltpu.SemaphoreType.DMA,
        ],
    )
    return kern(stab, dtab, src, dst)


def _dense_body(gs, gd, ef8, we2, be2, wed, wef8, y, w8):
    a = gs[:, :HID]
    ks = gs[:, HID:2 * HID]
    bd = gd[:, :HID]
    qd = gd[:, HID:2 * HID]
    diff = gs[:, 2 * HID:2 * HID + 3] - gd[:, 2 * HID:2 * HID + 3]
    d2 = jnp.sum(diff * diff, axis=1, keepdims=True)
    dist = jnp.sqrt(d2)
    t1 = a + bd + dist * wed[...] + jnp.dot(
        ef8[...], wef8[...], preferred_element_type=jnp.float32)
    u = _silu(t1)
    mh = _silu(jnp.dot(u, we2[...], preferred_element_type=jnp.float32)
               + be2[...])
    e = (jnp.sum(ks * qd, axis=1, keepdims=True)
         + dist * gd[:, 2 * HID + 3:2 * HID + 4]
         + jnp.sum(ef8[:, :4] * gd[:, 2 * HID + 4:2 * HID + 8],
                   axis=1, keepdims=True)
         ) * (1.0 / math.sqrt(HID))
    w = jnp.exp(e)
    y[...] = mh * w
    zp = jnp.zeros((gs.shape[0], 7), jnp.float32)
    w8[...] = jnp.concatenate([w, zp], axis=1)


def _dense(gs, gd, ef8, we2, be2, wed, wef8):
    grid = (E // BE,)
    return pl.pallas_call(
        _dense_body,
        grid=grid,
        in_specs=[
            pl.BlockSpec((BE, TW), lambda i: (i, 0)),
            pl.BlockSpec((BE, TW), lambda i: (i, 0)),
            pl.BlockSpec((BE, 8), lambda i: (i, 0)),
            pl.BlockSpec((HID, HID), lambda i: (0, 0)),
            pl.BlockSpec((1, HID), lambda i: (0, 0)),
            pl.BlockSpec((1, HID), lambda i: (0, 0)),
            pl.BlockSpec((8, HID), lambda i: (0, 0)),
        ],
        out_specs=[
            pl.BlockSpec((BE, HID), lambda i: (i, 0)),
            pl.BlockSpec((BE, 8), lambda i: (i, 0)),
        ],
        out_shape=[
            jax.ShapeDtypeStruct((E, HID), jnp.float32),
            jax.ShapeDtypeStruct((E, 8), jnp.float32),
        ],
    )(gs, gd, ef8, we2, be2, wed, wef8)


def _scatter_kernel(y, w8, dst, znb, out, acc, ybuf, wbuf, ywbuf, idxb):
    cid = lax.axis_index("c")
    sid = lax.axis_index("s")
    row0 = sid * SC_ROWS_PER_SUB
    pltpu.sync_copy(znb.at[pl.ds(row0, SC_ROWS_PER_SUB)],
                    acc.at[pl.ds(row0, SC_ROWS_PER_SUB)])
    plsc.subcore_barrier()
    base0 = sid * (E // NS)
    nchunk = (E // NS) // SCC

    @pl.when(cid == 0)
    def _():
        @pl.loop(0, nchunk)
        def _(k):
            base = base0 + k * SCC
            pltpu.sync_copy(dst.at[pl.ds(base, SCC)], idxb)
            pltpu.sync_copy(y.at[pl.ds(base, SCC)], ybuf)
            pltpu.sync_copy(ybuf, acc.at[idxb], add=True)

    @pl.when(cid == 1)
    def _():
        pltpu.sync_copy(znb.at[pl.ds(0, SCC)], ywbuf)

        @pl.loop(0, nchunk)
        def _(k):
            base = base0 + k * SCC
            pltpu.sync_copy(dst.at[pl.ds(base, SCC)], idxb)
            pltpu.sync_copy(w8.at[pl.ds(base, SCC)], wbuf)

            @pl.loop(0, SCC // 16)
            def _(j):
                row16 = lax.iota(jnp.int32, 16) + j * 16
                zero16 = jnp.zeros((16,), jnp.int32)
                w16 = plsc.load_gather(wbuf, [row16, zero16])
                plsc.store_scatter(ywbuf, [row16, zero16], w16)

            pltpu.sync_copy(ywbuf, acc.at[idxb], add=True)

    plsc.subcore_barrier()
    pltpu.sync_copy(acc.at[pl.ds(row0, SC_ROWS_PER_SUB)],
                    out.at[cid, pl.ds(row0, SC_ROWS_PER_SUB)])


def _sc_params():
    cp = pltpu.CompilerParams()
    if "needs_layout_passes" in pltpu.CompilerParams.__dataclass_fields__:
        cp = dataclasses.replace(cp, needs_layout_passes=False)
    return cp


def _scatter(y, w8, dst, znb):
    mesh = plsc.VectorSubcoreMesh(core_axis_name="c", subcore_axis_name="s")
    kern = pl.kernel(
        _scatter_kernel,
        mesh=mesh,
        compiler_params=_sc_params(),
        out_type=jax.ShapeDtypeStruct((NC, NP, HID), jnp.float32),
        scratch_types=[
            pltpu.VMEM_SHARED((NP, HID), jnp.float32),
            pltpu.VMEM((SCC, HID), jnp.float32),
            pltpu.VMEM((SCC, 8), jnp.float32),
            pltpu.VMEM((SCC, HID), jnp.float32),
            pltpu.VMEM((SCC,), jnp.int32),
        ],
    )
    return kern(y, w8, dst, znb)


def _final_body(nf, p0, p1, wn1a, wn1b, bn1, wn2, bn2, h):
    num = p0[0]
    den = p1[0, :, 0:1]
    den = jnp.where(den == 0.0, 1.0, den)
    hn = num / den
    pre = (jnp.dot(nf[...], wn1a[...], preferred_element_type=jnp.float32)
           + jnp.dot(hn, wn1b[...], preferred_element_type=jnp.float32)
           + bn1[...])
    h[...] = (jnp.dot(_silu(pre), wn2[...],
                      preferred_element_type=jnp.float32) + bn2[...])


def _final(nf, parts, wn1a, wn1b, bn1, wn2, bn2):
    grid = (N // RB,)
    return pl.pallas_call(
        _final_body,
        grid=grid,
        in_specs=[
            pl.BlockSpec((RB, IN), lambda i: (i, 0)),
            pl.BlockSpec((1, RB, HID), lambda i: (0, i, 0)),
            pl.BlockSpec((1, RB, HID), lambda i: (1, i, 0)),
            pl.BlockSpec((IN, HID), lambda i: (0, 0)),
            pl.BlockSpec((HID, HID), lambda i: (0, 0)),
            pl.BlockSpec((1, HID), lambda i: (0, 0)),
            pl.BlockSpec((HID, OUT), lambda i: (0, 0)),
            pl.BlockSpec((1, OUT), lambda i: (0, 0)),
        ],
        out_specs=pl.BlockSpec((RB, OUT), lambda i: (i, 0)),
        out_shape=jax.ShapeDtypeStruct((N, OUT), jnp.float32),
    )(nf, parts, parts, wn1a, wn1b, bn1, wn2, bn2)


@jax.jit
def kernel(node_feat, coord_feat, edge_index, edge_feat,
           W_e1, b_e1, W_e2, b_e2, W_q, W_k, W_n1, b_n1, W_n2, b_n2):
    src = edge_index[0]
    dst = edge_index[1]
    wcat = jnp.concatenate(
        [W_e1[:IN], W_e1[IN:2 * IN], W_k[:IN], W_q], axis=1)
    wt8 = jnp.pad(W_k[IN:].T, ((0, 0), (0, 3)))
    wed = W_e1[2 * IN:2 * IN + 1]
    wef8 = jnp.pad(W_e1[2 * IN + 1:], ((0, 4), (0, 0)))
    coord8 = jnp.pad(coord_feat, ((0, 0), (0, 5)))
    ef8 = jnp.pad(edge_feat, ((0, 0), (0, 4)))
    be1 = b_e1.reshape(1, HID)
    be2 = b_e2.reshape(1, HID)
    bn1 = b_n1.reshape(1, HID)
    bn2 = b_n2.reshape(1, OUT)
    znb = jnp.zeros((NP, HID), jnp.float32)

    stab, dtab = _make_tables(node_feat, coord8, wcat, be1, wt8)
    gs, gd = _gather(stab, dtab, src, dst)
    y, w8 = _dense(gs, gd, ef8, W_e2, be2, wed, wef8)
    parts = _scatter(y, w8, dst, znb)
    return _final(node_feat, parts, W_n1[:IN], W_n1[IN:], bn1, W_n2, bn2)


# speedup vs baseline: 3.8804x; 1.1269x over previous
import dataclasses
import functools
import math

import jax
import jax.numpy as jnp
from jax import lax
from jax.experimental import pallas as pl
from jax.experimental.pallas import tpu as pltpu
from jax.experimental.pallas import tpu_sc as plsc

N = 10000
E = 320000
IN = 128
HID = 128
OUT = 128
EF = 4

NC = 2
NS = 16
NW = NC * NS
EPW = E // NW
GC = 80
NCHUNK = EPW // GC
GV = GC // 16
NP = 10240
SC_ROWS_PER_SUB = NP // NS

TW = 256
SCC = 80

RB = 400
BE = 640


def _silu(x):
    return x * jax.nn.sigmoid(x)


def _tables_body(nf, wcat, be1, stab, dtab):
    m = jnp.dot(nf[...], wcat[...], preferred_element_type=jnp.float32)
    a = m[:, :HID] + be1[...]
    b = m[:, HID:2 * HID]
    k = m[:, 2 * HID:3 * HID]
    q = m[:, 3 * HID:]
    stab[...] = jnp.concatenate([a, k], axis=1)
    dtab[...] = jnp.concatenate([b, q], axis=1)


def _make_tables(nf, wcat, be1):
    grid = (N // RB,)
    return pl.pallas_call(
        _tables_body,
        grid=grid,
        in_specs=[
            pl.BlockSpec((RB, IN), lambda i: (i, 0)),
            pl.BlockSpec((IN, 4 * HID), lambda i: (0, 0)),
            pl.BlockSpec((1, HID), lambda i: (0, 0)),
        ],
        out_specs=[
            pl.BlockSpec((RB, TW), lambda i: (i, 0)),
            pl.BlockSpec((RB, TW), lambda i: (i, 0)),
        ],
        out_shape=[
            jax.ShapeDtypeStruct((N, TW), jnp.float32),
            jax.ShapeDtypeStruct((N, TW), jnp.float32),
        ],
    )(nf, wcat, be1)


def _gather_kernel(stab, dtab, src, dst, cx, cy, cz,
                   gs, gd, x8,
                   idxs, idxd, rs, rd, xb, cxb, cyb, czb, s0, s1):
    wid = lax.axis_index("s") * NC + lax.axis_index("c")
    pltpu.sync_copy(cx, cxb)
    pltpu.sync_copy(cy, cyb)
    pltpu.sync_copy(cz, czb)

    @pl.loop(0, NCHUNK)
    def _(k):
        base = wid * EPW + k * GC
        pltpu.sync_copy(src.at[pl.ds(base, GC)], idxs)
        pltpu.sync_copy(dst.at[pl.ds(base, GC)], idxd)
        c0 = pltpu.async_copy(stab.at[idxs], rs, s0)
        c1 = pltpu.async_copy(dtab.at[idxd], rd, s1)

        @pl.loop(0, GC // 16)
        def _(j):
            pos = lax.iota(jnp.int32, 16) + j * 16
            zero16 = jnp.zeros((16,), jnp.int32)
            si = plsc.load_gather(idxs, [pos])
            di = plsc.load_gather(idxd, [pos])
            dx = plsc.load_gather(cxb, [si]) - plsc.load_gather(cxb, [di])
            dy = plsc.load_gather(cyb, [si]) - plsc.load_gather(cyb, [di])
            dz = plsc.load_gather(czb, [si]) - plsc.load_gather(czb, [di])
            d2 = dx * dx + dy * dy + dz * dz
            plsc.store_scatter(xb, [pos, zero16], d2)

        c0.wait()
        c1.wait()
        pltpu.sync_copy(rs, gs.at[pl.ds(base, GC)])
        pltpu.sync_copy(rd, gd.at[pl.ds(base, GC)])
        pltpu.sync_copy(xb, x8.at[pl.ds(base, GC)])


def _gather(stab, dtab, src, dst, cx, cy, cz):
    mesh = plsc.VectorSubcoreMesh(core_axis_name="c", subcore_axis_name="s")
    kern = pl.kernel(
        _gather_kernel,
        mesh=mesh,
        compiler_params=_sc_params(),
        out_type=[
            jax.ShapeDtypeStruct((E, TW), jnp.float32),
            jax.ShapeDtypeStruct((E, TW), jnp.float32),
            jax.ShapeDtypeStruct((E, 8), jnp.float32),
        ],
        scratch_types=[
            pltpu.VMEM((GC,), jnp.int32),
            pltpu.VMEM((GC,), jnp.int32),
            pltpu.VMEM((GC, TW), jnp.float32),
            pltpu.VMEM((GC, TW), jnp.float32),
            pltpu.VMEM((GC, 8), jnp.float32),
            pltpu.VMEM((NP,), jnp.float32),
            pltpu.VMEM((NP,), jnp.float32),
            pltpu.VMEM((NP,), jnp.float32),
            pltpu.SemaphoreType.DMA,
            pltpu.SemaphoreType.DMA,
        ],
    )
    return kern(stab, dtab, src, dst, cx, cy, cz)


def _dense_body(gs, gd, x8, ef8, we2, be2, wed, wef8, wkd, wkef8, y, w8):
    a = gs[:, :HID]
    ks = gs[:, HID:]
    bd = gd[:, :HID]
    qd = gd[:, HID:]
    dist = jnp.sqrt(x8[:, 0:1])
    efm = jnp.dot(ef8[...], wef8[...], preferred_element_type=jnp.float32)
    t1 = a + bd + dist * wed[...] + efm
    u = _silu(t1)
    mh = _silu(jnp.dot(u, we2[...], preferred_element_type=jnp.float32)
               + be2[...])
    mk = ks + dist * wkd[...] + jnp.dot(
        ef8[...], wkef8[...], preferred_element_type=jnp.float32)
    e = jnp.sum(mk * qd, axis=1, keepdims=True) * (1.0 / math.sqrt(HID))
    w = jnp.exp(e)
    y[...] = mh * w
    zp = jnp.zeros((gs.shape[0], 7), jnp.float32)
    w8[...] = jnp.concatenate([w, zp], axis=1)


def _dense(gs, gd, x8, ef8, we2, be2, wed, wef8, wkd, wkef8):
    grid = (E // BE,)
    return pl.pallas_call(
        _dense_body,
        grid=grid,
        in_specs=[
            pl.BlockSpec((BE, TW), lambda i: (i, 0)),
            pl.BlockSpec((BE, TW), lambda i: (i, 0)),
            pl.BlockSpec((BE, 8), lambda i: (i, 0)),
            pl.BlockSpec((BE, 8), lambda i: (i, 0)),
            pl.BlockSpec((HID, HID), lambda i: (0, 0)),
            pl.BlockSpec((1, HID), lambda i: (0, 0)),
            pl.BlockSpec((1, HID), lambda i: (0, 0)),
            pl.BlockSpec((8, HID), lambda i: (0, 0)),
            pl.BlockSpec((1, HID), lambda i: (0, 0)),
            pl.BlockSpec((8, HID), lambda i: (0, 0)),
        ],
        out_specs=[
            pl.BlockSpec((BE, HID), lambda i: (i, 0)),
            pl.BlockSpec((BE, 8), lambda i: (i, 0)),
        ],
        out_shape=[
            jax.ShapeDtypeStruct((E, HID), jnp.float32),
            jax.ShapeDtypeStruct((E, 8), jnp.float32),
        ],
    )(gs, gd, x8, ef8, we2, be2, wed, wef8, wkd, wkef8)


def _scatter_kernel(y, w8, dst, znb, out, acc, ybuf, wbuf, ywbuf, idxb):
    cid = lax.axis_index("c")
    sid = lax.axis_index("s")
    row0 = sid * SC_ROWS_PER_SUB
    pltpu.sync_copy(znb.at[pl.ds(row0, SC_ROWS_PER_SUB)],
                    acc.at[pl.ds(row0, SC_ROWS_PER_SUB)])
    plsc.subcore_barrier()
    base0 = sid * (E // NS)
    nchunk = (E // NS) // SCC

    @pl.when(cid == 0)
    def _():
        @pl.loop(0, nchunk)
        def _(k):
            base = base0 + k * SCC
            pltpu.sync_copy(dst.at[pl.ds(base, SCC)], idxb)
            pltpu.sync_copy(y.at[pl.ds(base, SCC)], ybuf)
            pltpu.sync_copy(ybuf, acc.at[idxb], add=True)

    @pl.when(cid == 1)
    def _():
        pltpu.sync_copy(znb.at[pl.ds(0, SCC)], ywbuf)

        @pl.loop(0, nchunk)
        def _(k):
            base = base0 + k * SCC
            pltpu.sync_copy(dst.at[pl.ds(base, SCC)], idxb)
            pltpu.sync_copy(w8.at[pl.ds(base, SCC)], wbuf)

            @pl.loop(0, SCC // 16)
            def _(j):
                row16 = lax.iota(jnp.int32, 16) + j * 16
                zero16 = jnp.zeros((16,), jnp.int32)
                w16 = plsc.load_gather(wbuf, [row16, zero16])
                plsc.store_scatter(ywbuf, [row16, zero16], w16)

            pltpu.sync_copy(ywbuf, acc.at[idxb], add=True)

    plsc.subcore_barrier()
    pltpu.sync_copy(acc.at[pl.ds(row0, SC_ROWS_PER_SUB)],
                    out.at[cid, pl.ds(row0, SC_ROWS_PER_SUB)])


def _sc_params():
    cp = pltpu.CompilerParams()
    if "needs_layout_passes" in pltpu.CompilerParams.__dataclass_fields__:
        cp = dataclasses.replace(cp, needs_layout_passes=False)
    return cp


def _scatter(y, w8, dst, znb):
    mesh = plsc.VectorSubcoreMesh(core_axis_name="c", subcore_axis_name="s")
    kern = pl.kernel(
        _scatter_kernel,
        mesh=mesh,
        compiler_params=_sc_params(),
        out_type=jax.ShapeDtypeStruct((NC, NP, HID), jnp.float32),
        scratch_types=[
            pltpu.VMEM_SHARED((NP, HID), jnp.float32),
            pltpu.VMEM((SCC, HID), jnp.float32),
            pltpu.VMEM((SCC, 8), jnp.float32),
            pltpu.VMEM((SCC, HID), jnp.float32),
            pltpu.VMEM((SCC,), jnp.int32),
        ],
    )
    return kern(y, w8, dst, znb)


def _final_body(nf, p0, p1, wn1a, wn1b, bn1, wn2, bn2, h):
    num = p0[0]
    den = p1[0, :, 0:1]
    den = jnp.where(den == 0.0, 1.0, den)
    hn = num / den
    pre = (jnp.dot(nf[...], wn1a[...], preferred_element_type=jnp.float32)
           + jnp.dot(hn, wn1b[...], preferred_element_type=jnp.float32)
           + bn1[...])
    h[...] = (jnp.dot(_silu(pre), wn2[...],
                      preferred_element_type=jnp.float32) + bn2[...])


def _final(nf, parts, wn1a, wn1b, bn1, wn2, bn2):
    grid = (N // RB,)
    return pl.pallas_call(
        _final_body,
        grid=grid,
        in_specs=[
            pl.BlockSpec((RB, IN), lambda i: (i, 0)),
            pl.BlockSpec((1, RB, HID), lambda i: (0, i, 0)),
            pl.BlockSpec((1, RB, HID), lambda i: (1, i, 0)),
            pl.BlockSpec((IN, HID), lambda i: (0, 0)),
            pl.BlockSpec((HID, HID), lambda i: (0, 0)),
            pl.BlockSpec((1, HID), lambda i: (0, 0)),
            pl.BlockSpec((HID, OUT), lambda i: (0, 0)),
            pl.BlockSpec((1, OUT), lambda i: (0, 0)),
        ],
        out_specs=pl.BlockSpec((RB, OUT), lambda i: (i, 0)),
        out_shape=jax.ShapeDtypeStruct((N, OUT), jnp.float32),
    )(nf, parts, parts, wn1a, wn1b, bn1, wn2, bn2)


@jax.jit
def kernel(node_feat, coord_feat, edge_index, edge_feat,
           W_e1, b_e1, W_e2, b_e2, W_q, W_k, W_n1, b_n1, W_n2, b_n2):
    src = edge_index[0]
    dst = edge_index[1]
    wcat = jnp.concatenate(
        [W_e1[:IN], W_e1[IN:2 * IN], W_k[:IN], W_q], axis=1)
    wed = W_e1[2 * IN:2 * IN + 1]
    wef8 = jnp.pad(W_e1[2 * IN + 1:], ((0, 4), (0, 0)))
    wkd = W_k[IN:IN + 1]
    wkef8 = jnp.pad(W_k[IN + 1:], ((0, 4), (0, 0)))
    cpad = jnp.pad(coord_feat, ((0, NP - N), (0, 0)))
    ef8 = jnp.pad(edge_feat, ((0, 0), (0, 4)))
    be1 = b_e1.reshape(1, HID)
    be2 = b_e2.reshape(1, HID)
    bn1 = b_n1.reshape(1, HID)
    bn2 = b_n2.reshape(1, OUT)
    znb = jnp.zeros((NP, HID), jnp.float32)

    stab, dtab = _make_tables(node_feat, wcat, be1)
    gs, gd, x8 = _gather(stab, dtab, src, dst,
                         cpad[:, 0], cpad[:, 1], cpad[:, 2])
    y, w8 = _dense(gs, gd, x8, ef8, W_e2, be2, wed, wef8, wkd, wkef8)
    parts = _scatter(y, w8, dst, znb)
    return _final(node_feat, parts, W_n1[:IN], W_n1[IN:], bn1, W_n2, bn2)


# speedup vs baseline: 4.7833x; 1.2327x over previous
import dataclasses
import functools
import math

import jax
import jax.numpy as jnp
from jax import lax
from jax.experimental import pallas as pl
from jax.experimental.pallas import tpu as pltpu
from jax.experimental.pallas import tpu_sc as plsc

N = 10000
E = 320000
IN = 128
HID = 128
OUT = 128
EF = 4

NC = 2
NS = 16
NW = NC * NS
EPW = E // NW
GC = 80
NCHUNK = EPW // GC
GV = GC // 16
NP = 10240
SC_ROWS_PER_SUB = NP // NS

TW = 256
TWP = 128
SCC = 80

RB = 400
RB2 = 512
BE = 640


def _silu(x):
    return x * jax.nn.sigmoid(x)


def _tables_body(nf, wcat, be1, stab, dtab):
    m = jnp.dot(nf[...], wcat[...], preferred_element_type=jnp.float32)
    a = m[:, :HID] + be1[...]
    b = m[:, HID:2 * HID]
    k = m[:, 2 * HID:3 * HID]
    q = m[:, 3 * HID:]
    def pack(lo, hi):
        lou = lax.bitcast_convert_type(
            lo.astype(jnp.bfloat16).astype(jnp.float32), jnp.uint32)
        hiu = lax.bitcast_convert_type(
            hi.astype(jnp.bfloat16).astype(jnp.float32), jnp.uint32)
        return lax.bitcast_convert_type(
            (lou >> 16) | (hiu & jnp.uint32(0xFFFF0000)), jnp.int32)

    stab[...] = pack(a, k)
    dtab[...] = pack(b, q)


def _make_tables(nf, wcat, be1):
    grid = (N // RB,)
    return pl.pallas_call(
        _tables_body,
        grid=grid,
        in_specs=[
            pl.BlockSpec((RB, IN), lambda i: (i, 0)),
            pl.BlockSpec((IN, 4 * HID), lambda i: (0, 0)),
            pl.BlockSpec((1, HID), lambda i: (0, 0)),
        ],
        out_specs=[
            pl.BlockSpec((RB, TWP), lambda i: (i, 0)),
            pl.BlockSpec((RB, TWP), lambda i: (i, 0)),
        ],
        out_shape=[
            jax.ShapeDtypeStruct((N, TWP), jnp.int32),
            jax.ShapeDtypeStruct((N, TWP), jnp.int32),
        ],
    )(nf, wcat, be1)


def _gather_kernel(stab, dtab, src, dst, cx, cy, cz,
                   gs, gd, x8,
                   idxs, idxd, rs, rd, xb, cxb, cyb, czb, s0, s1):
    wid = lax.axis_index("s") * NC + lax.axis_index("c")
    pltpu.sync_copy(cx, cxb)
    pltpu.sync_copy(cy, cyb)
    pltpu.sync_copy(cz, czb)

    @pl.loop(0, NCHUNK)
    def _(k):
        base = wid * EPW + k * GC
        pltpu.sync_copy(src.at[pl.ds(base, GC)], idxs)
        pltpu.sync_copy(dst.at[pl.ds(base, GC)], idxd)
        c0 = pltpu.async_copy(stab.at[idxs], rs, s0)
        c1 = pltpu.async_copy(dtab.at[idxd], rd, s1)

        @pl.loop(0, GC // 16)
        def _(j):
            pos = lax.iota(jnp.int32, 16) + j * 16
            zero16 = jnp.zeros((16,), jnp.int32)
            si = plsc.load_gather(idxs, [pos])
            di = plsc.load_gather(idxd, [pos])
            dx = plsc.load_gather(cxb, [si]) - plsc.load_gather(cxb, [di])
            dy = plsc.load_gather(cyb, [si]) - plsc.load_gather(cyb, [di])
            dz = plsc.load_gather(czb, [si]) - plsc.load_gather(czb, [di])
            d2 = dx * dx + dy * dy + dz * dz
            plsc.store_scatter(xb, [pos, zero16], d2)

        c0.wait()
        c1.wait()
        pltpu.sync_copy(rs, gs.at[pl.ds(base, GC)])
        pltpu.sync_copy(rd, gd.at[pl.ds(base, GC)])
        pltpu.sync_copy(xb, x8.at[pl.ds(base, GC)])


def _gather(stab, dtab, src, dst, cx, cy, cz):
    mesh = plsc.VectorSubcoreMesh(core_axis_name="c", subcore_axis_name="s")
    kern = pl.kernel(
        _gather_kernel,
        mesh=mesh,
        compiler_params=_sc_params(),
        out_type=[
            jax.ShapeDtypeStruct((E, TWP), jnp.int32),
            jax.ShapeDtypeStruct((E, TWP), jnp.int32),
            jax.ShapeDtypeStruct((E, 8), jnp.float32),
        ],
        scratch_types=[
            pltpu.VMEM((GC,), jnp.int32),
            pltpu.VMEM((GC,), jnp.int32),
            pltpu.VMEM((GC, TWP), jnp.int32),
            pltpu.VMEM((GC, TWP), jnp.int32),
            pltpu.VMEM((GC, 8), jnp.float32),
            pltpu.VMEM((NP,), jnp.float32),
            pltpu.VMEM((NP,), jnp.float32),
            pltpu.VMEM((NP,), jnp.float32),
            pltpu.SemaphoreType.DMA,
            pltpu.SemaphoreType.DMA,
        ],
    )
    return kern(stab, dtab, src, dst, cx, cy, cz)


def _dense_body(gs, gd, x8, ef8, we2, be2, wed, wef8, wkd, wkef8, y, w8):
    gu = lax.bitcast_convert_type(gs[...], jnp.uint32)
    du = lax.bitcast_convert_type(gd[...], jnp.uint32)
    a = lax.bitcast_convert_type(gu << 16, jnp.float32)
    ks = lax.bitcast_convert_type(gu & jnp.uint32(0xFFFF0000), jnp.float32)
    bd = lax.bitcast_convert_type(du << 16, jnp.float32)
    qd = lax.bitcast_convert_type(du & jnp.uint32(0xFFFF0000), jnp.float32)
    dist = jnp.sqrt(x8[:, 0:1])
    efm = jnp.dot(ef8[...], wef8[...], preferred_element_type=jnp.float32)
    t1 = a + bd + dist * wed[...] + efm
    u = _silu(t1)
    mh = _silu(jnp.dot(u, we2[...], preferred_element_type=jnp.float32)
               + be2[...])
    mk = ks + dist * wkd[...] + jnp.dot(
        ef8[...], wkef8[...], preferred_element_type=jnp.float32)
    e = jnp.sum(mk * qd, axis=1, keepdims=True) * (1.0 / math.sqrt(HID))
    w = jnp.exp(e)
    y[...] = mh * w
    zp = jnp.zeros((gs.shape[0], 7), jnp.float32)
    w8[...] = jnp.concatenate([w, zp], axis=1)


def _dense(gs, gd, x8, ef8, we2, be2, wed, wef8, wkd, wkef8):
    grid = (E // BE,)
    return pl.pallas_call(
        _dense_body,
        grid=grid,
        in_specs=[
            pl.BlockSpec((BE, TWP), lambda i: (i, 0)),
            pl.BlockSpec((BE, TWP), lambda i: (i, 0)),
            pl.BlockSpec((BE, 8), lambda i: (i, 0)),
            pl.BlockSpec((BE, 8), lambda i: (i, 0)),
            pl.BlockSpec((HID, HID), lambda i: (0, 0)),
            pl.BlockSpec((1, HID), lambda i: (0, 0)),
            pl.BlockSpec((1, HID), lambda i: (0, 0)),
            pl.BlockSpec((8, HID), lambda i: (0, 0)),
            pl.BlockSpec((1, HID), lambda i: (0, 0)),
            pl.BlockSpec((8, HID), lambda i: (0, 0)),
        ],
        out_specs=[
            pl.BlockSpec((BE, HID), lambda i: (i, 0)),
            pl.BlockSpec((BE, 8), lambda i: (i, 0)),
        ],
        out_shape=[
            jax.ShapeDtypeStruct((E, HID), jnp.float32),
            jax.ShapeDtypeStruct((E, 8), jnp.float32),
        ],
    )(gs, gd, x8, ef8, we2, be2, wed, wef8, wkd, wkef8)


def _scatter_kernel(y, w8, dst, znb, zn1, out, dout, acc, ybuf, wbuf, idxb,
                    denb):
    cid = lax.axis_index("c")
    sid = lax.axis_index("s")
    row0 = sid * SC_ROWS_PER_SUB
    pltpu.sync_copy(znb.at[pl.ds(row0, SC_ROWS_PER_SUB)],
                    acc.at[pl.ds(row0, SC_ROWS_PER_SUB)])
    pltpu.sync_copy(zn1, denb)
    plsc.subcore_barrier()
    half = E // NC
    per_sub = half // NS
    nchunk = per_sub // SCC
    base0 = cid * half + sid * per_sub

    @pl.loop(0, nchunk)
    def _(k):
        base = base0 + k * SCC
        pltpu.sync_copy(dst.at[pl.ds(base, SCC)], idxb)
        pltpu.sync_copy(y.at[pl.ds(base, SCC)], ybuf)
        pltpu.sync_copy(w8.at[pl.ds(base, SCC)], wbuf)
        pltpu.sync_copy(ybuf, acc.at[idxb], add=True)

        @pl.loop(0, SCC // 16)
        def _(j):
            pos = lax.iota(jnp.int32, 16) + j * 16
            zero16 = jnp.zeros((16,), jnp.int32)
            dv = plsc.load_gather(idxb, [pos])
            wv = plsc.load_gather(wbuf, [pos, zero16])
            plsc.addupdate_scatter(denb, [dv], wv)

    plsc.subcore_barrier()
    pltpu.sync_copy(acc.at[pl.ds(row0, SC_ROWS_PER_SUB)],
                    out.at[cid, pl.ds(row0, SC_ROWS_PER_SUB)])
    pltpu.sync_copy(denb, dout.at[pl.ds((cid * NS + sid) * NP, NP)])


def _sc_params():
    cp = pltpu.CompilerParams()
    if "needs_layout_passes" in pltpu.CompilerParams.__dataclass_fields__:
        cp = dataclasses.replace(cp, needs_layout_passes=False)
    return cp


def _scatter(y, w8, dst, znb, zn1):
    mesh = plsc.VectorSubcoreMesh(core_axis_name="c", subcore_axis_name="s")
    kern = pl.kernel(
        _scatter_kernel,
        mesh=mesh,
        compiler_params=_sc_params(),
        out_type=[
            jax.ShapeDtypeStruct((NC, NP, HID), jnp.float32),
            jax.ShapeDtypeStruct((NC * NS * NP,), jnp.float32),
        ],
        scratch_types=[
            pltpu.VMEM_SHARED((NP, HID), jnp.float32),
            pltpu.VMEM((SCC, HID), jnp.float32),
            pltpu.VMEM((SCC, 8), jnp.float32),
            pltpu.VMEM((SCC,), jnp.int32),
            pltpu.VMEM((NP,), jnp.float32),
        ],
    )
    return kern(y, w8, dst, znb, zn1)


def _final_body(nf, p0, p1, d32, wn1a, wn1b, bn1, wn2, bn2, h):
    num = p0[0] + p1[0]
    den = jnp.sum(d32[...], axis=0)[:, None]
    den = jnp.where(den == 0.0, 1.0, den)
    hn = num / den
    pre = (jnp.dot(nf[...], wn1a[...], preferred_element_type=jnp.float32)
           + jnp.dot(hn, wn1b[...], preferred_element_type=jnp.float32)
           + bn1[...])
    h[...] = (jnp.dot(_silu(pre), wn2[...],
                      preferred_element_type=jnp.float32) + bn2[...])


def _final(nfp, parts, d32, wn1a, wn1b, bn1, wn2, bn2):
    grid = (NP // RB2,)
    return pl.pallas_call(
        _final_body,
        grid=grid,
        in_specs=[
            pl.BlockSpec((RB2, IN), lambda i: (i, 0)),
            pl.BlockSpec((1, RB2, HID), lambda i: (0, i, 0)),
            pl.BlockSpec((1, RB2, HID), lambda i: (1, i, 0)),
            pl.BlockSpec((NW, RB2), lambda i: (0, i)),
            pl.BlockSpec((IN, HID), lambda i: (0, 0)),
            pl.BlockSpec((HID, HID), lambda i: (0, 0)),
            pl.BlockSpec((1, HID), lambda i: (0, 0)),
            pl.BlockSpec((HID, OUT), lambda i: (0, 0)),
            pl.BlockSpec((1, OUT), lambda i: (0, 0)),
        ],
        out_specs=pl.BlockSpec((RB2, OUT), lambda i: (i, 0)),
        out_shape=jax.ShapeDtypeStruct((NP, OUT), jnp.float32),
    )(nfp, parts, parts, d32, wn1a, wn1b, bn1, wn2, bn2)


@jax.jit
def kernel(node_feat, coord_feat, edge_index, edge_feat,
           W_e1, b_e1, W_e2, b_e2, W_q, W_k, W_n1, b_n1, W_n2, b_n2):
    src = edge_index[0]
    dst = edge_index[1]
    wcat = jnp.concatenate(
        [W_e1[:IN], W_e1[IN:2 * IN], W_k[:IN], W_q], axis=1)
    wed = W_e1[2 * IN:2 * IN + 1]
    wef8 = jnp.pad(W_e1[2 * IN + 1:], ((0, 4), (0, 0)))
    wkd = W_k[IN:IN + 1]
    wkef8 = jnp.pad(W_k[IN + 1:], ((0, 4), (0, 0)))
    cpad = jnp.pad(coord_feat, ((0, NP - N), (0, 0)))
    ef8 = jnp.pad(edge_feat, ((0, 0), (0, 4)))
    be1 = b_e1.reshape(1, HID)
    be2 = b_e2.reshape(1, HID)
    bn1 = b_n1.reshape(1, HID)
    bn2 = b_n2.reshape(1, OUT)
    znb = jnp.zeros((NP, HID), jnp.float32)
    zn1 = jnp.zeros((NP,), jnp.float32)

    stab, dtab = _make_tables(node_feat, wcat, be1)
    gs, gd, x8 = _gather(stab, dtab, src, dst,
                         cpad[:, 0], cpad[:, 1], cpad[:, 2])
    y, w8 = _dense(gs, gd, x8, ef8, W_e2, be2, wed, wef8, wkd, wkef8)
    parts, dflat = _scatter(y, w8, dst, znb, zn1)
    d32 = dflat.reshape(NW, NP)
    nfp = jnp.pad(node_feat, ((0, NP - N), (0, 0)))
    h = _final(nfp, parts, d32, W_n1[:IN], W_n1[IN:], bn1, W_n2, bn2)
    return h[:N]
